# baseline (speedup 1.0000x reference)
import functools

import jax
import jax.numpy as jnp
from jax import lax
from jax.experimental import pallas as pl
from jax.experimental.pallas import tpu as pltpu
from jax.experimental.pallas import tpu_sc as plsc

_N_USERS = 4000
_N_ITEMS = 6000
_N = _N_USERS + _N_ITEMS
_E = 320000
_D = 128
_HOPS = 3

_NC = 2
_NS = 16
_NW = _NC * _NS
_C = 128
_NBUF = 2
_G = 8
_NG = 10
_NCH = _G * _NG
_EPAD = _NW * _NCH * _C
_RPS = 632
_NPAD = _NS * _RPS

_mesh = plsc.VectorSubcoreMesh(core_axis_name="core", subcore_axis_name="subcore")


@functools.partial(
    pl.kernel,
    mesh=_mesh,
    out_type=jax.ShapeDtypeStruct((_NC, _NPAD, _D), jnp.float32),
    scratch_types=[
        pltpu.VMEM_SHARED((_NPAD, _D), jnp.float32),
        pltpu.VMEM((2, _G, _C), jnp.int32),
        pltpu.VMEM((2, _G, _C), jnp.int32),
        pltpu.VMEM((2, _G, _C), jnp.float32),
        pltpu.VMEM((_NBUF, _C, _D), jnp.float32),
        pltpu.SemaphoreType.DMA((_NBUF,)),
        pltpu.SemaphoreType.DMA((_NBUF,)),
        pltpu.SemaphoreType.DMA((2,)),
    ],
)
def _hop(src_hbm, col_hbm, row_hbm, val_hbm, out_hbm,
         acc, col_v, row_v, val_v, rows_v, gsem, ssem, isem):
    c = lax.axis_index("core")
    s = lax.axis_index("subcore")
    wid = c * _NS + s

    def stage_group(slot, g, issue):
        srcs = (col_hbm, row_hbm, val_hbm)
        dsts = (col_v, row_v, val_v)
        for src, dst in zip(srcs, dsts):
            cp = pltpu.make_async_copy(src.at[wid, pl.ds(g * _G, _G)],
                                       dst.at[slot], isem.at[slot])
            if issue:
                cp.start()
            else:
                cp.wait()

    @pl.loop(0, _C)
    def _(i):
        @pl.loop(0, _D, step=16)
        def _(j):
            rows_v[0, i, pl.ds(j, 16)] = jnp.zeros((16,), jnp.float32)

    for k in range(4):
        pltpu.sync_copy(rows_v.at[0], acc.at[pl.ds(s * _RPS + k * _C, _C)])
    pltpu.sync_copy(rows_v.at[0, pl.ds(0, _RPS - 4 * _C)],
                    acc.at[pl.ds(s * _RPS + 4 * _C, _RPS - 4 * _C)])

    stage_group(0, 0, issue=True)
    stage_group(0, 0, issue=False)
    stage_group(1, 1, issue=True)

    plsc.subcore_barrier()

    for b in range(_NBUF):
        pltpu.async_copy(src_hbm.at[col_v.at[0, b]], rows_v.at[b], gsem.at[b])

    @pl.loop(0, _NG, step=2)
    def _(g0):
        for gp in range(2):
            g = g0 + gp
            for jg in range(_G):
                b = jg % _NBUF
                pltpu.make_async_copy(src_hbm.at[col_v.at[gp, jg]],
                                      rows_v.at[b], gsem.at[b]).wait()

                @pl.loop(0, _C, step=16)
                def _(e0):
                    vv = val_v[gp, jg, pl.ds(e0, 16)]
                    for l in range(16):
                        v = vv[l]
                        for sub in range(_D // 16):
                            sl = pl.ds(sub * 16, 16)
                            rows_v[b, e0 + l, sl] = rows_v[b, e0 + l, sl] * v

                pltpu.async_copy(rows_v.at[b], acc.at[row_v.at[gp, jg]],
                                 ssem.at[b], add=True).wait()

                if jg == _G - _NBUF:
                    @pl.when(g + 1 < _NG)
                    def _():
                        stage_group(1 - gp, g + 1, issue=False)

                nxt = jg + _NBUF
                if nxt < _G:
                    pltpu.async_copy(src_hbm.at[col_v.at[gp, nxt]],
                                     rows_v.at[b], gsem.at[b])
                else:
                    @pl.when(g + 1 < _NG)
                    def _():
                        pltpu.async_copy(
                            src_hbm.at[col_v.at[1 - gp, nxt - _G]],
                            rows_v.at[b], gsem.at[b])

            @pl.when(g + 2 < _NG)
            def _():
                stage_group(gp, g + 2, issue=True)

    plsc.subcore_barrier()

    pltpu.sync_copy(acc.at[pl.ds(s * _RPS, _RPS)],
                    out_hbm.at[c, pl.ds(s * _RPS, _RPS)])


def _add_body(p_ref, o_ref):
    o_ref[...] = p_ref[0] + p_ref[1]


_BLK = 632


def _combine(parts):
    return pl.pallas_call(
        _add_body,
        grid=(_NPAD // _BLK,),
        in_specs=[pl.BlockSpec((_NC, _BLK, _D), lambda i: (0, i, 0))],
        out_specs=pl.BlockSpec((_BLK, _D), lambda i: (i, 0)),
        out_shape=jax.ShapeDtypeStruct((_NPAD, _D), jnp.float32),
    )(parts)


def kernel(user_embed, item_embed, adj_indices, adj_values):
    x = jnp.concatenate([user_embed, item_embed], axis=0)
    pad = _EPAD - _E
    row = jnp.concatenate([adj_indices[0], jnp.zeros((pad,), jnp.int32)])
    col = jnp.concatenate([adj_indices[1], jnp.zeros((pad,), jnp.int32)])
    val = jnp.concatenate([adj_values, jnp.zeros((pad,), jnp.float32)])
    row = row.reshape(_NW, _NCH, _C)
    col = col.reshape(_NW, _NCH, _C)
    val = val.reshape(_NW, _NCH, _C)

    embs = [x]
    for _ in range(_HOPS):
        parts = _hop(x, col, row, val)
        x = _combine(parts)[: _N]
        embs.append(x)
    embs = jnp.stack(embs, axis=1)
    return embs[:_N_USERS], embs[_N_USERS:]


# speedup vs baseline: 1.5501x; 1.5501x over previous
import dataclasses
import functools

import jax
import jax.numpy as jnp
from jax import lax
from jax.experimental import pallas as pl
from jax.experimental.pallas import tpu as pltpu
from jax.experimental.pallas import tpu_sc as plsc

_N_USERS = 4000
_N_ITEMS = 6000
_N = _N_USERS + _N_ITEMS
_E = 320000
_D = 128
_HOPS = 3

_NC = 2
_NS = 16
_NW = _NC * _NS
_C = 128
_NBUF = 2
_G = 8
_NG = 10
_NCH = _G * _NG
_EPAD = _NW * _NCH * _C
_RPS = 632
_NPAD = _NS * _RPS

_mesh = plsc.VectorSubcoreMesh(core_axis_name="core", subcore_axis_name="subcore")

_cp = pltpu.CompilerParams(use_tc_tiling_on_sc=False)
if "needs_layout_passes" in pltpu.CompilerParams.__dataclass_fields__:
    _cp = dataclasses.replace(_cp, needs_layout_passes=False)


@functools.partial(
    pl.kernel,
    mesh=_mesh,
    compiler_params=_cp,
    out_type=jax.ShapeDtypeStruct((_NC, _NPAD, _D), jnp.float32),
    scratch_types=[
        pltpu.VMEM_SHARED((_NPAD, _D), jnp.float32),
        pltpu.VMEM((2, _G, _C), jnp.int32),
        pltpu.VMEM((2, _G, _C), jnp.int32),
        pltpu.VMEM((2, _G, _C), jnp.float32),
        pltpu.VMEM((_NBUF, _C, _D // 2), jnp.int32),
        pltpu.VMEM((_C, _D), jnp.float32),
        pltpu.SemaphoreType.DMA((_NBUF,)),
        pltpu.SemaphoreType.DMA((_NBUF,)),
        pltpu.SemaphoreType.DMA((2,)),
    ],
)
def _hop(src_hbm, col_hbm, row_hbm, val_hbm, out_hbm,
         acc, col_v, row_v, val_v, rows_v, work_v, gsem, ssem, isem):
    c = lax.axis_index("core")
    s = lax.axis_index("subcore")
    wid = c * _NS + s

    def stage_group(slot, g, issue):
        srcs = (col_hbm, row_hbm, val_hbm)
        dsts = (col_v, row_v, val_v)
        for src, dst in zip(srcs, dsts):
            cp = pltpu.make_async_copy(src.at[wid, pl.ds(g * _G, _G)],
                                       dst.at[slot], isem.at[slot])
            if issue:
                cp.start()
            else:
                cp.wait()

    @pl.loop(0, _C)
    def _(i):
        @pl.loop(0, _D, step=16)
        def _(j):
            work_v[i, pl.ds(j, 16)] = jnp.zeros((16,), jnp.float32)

    for k in range(4):
        pltpu.sync_copy(work_v, acc.at[pl.ds(s * _RPS + k * _C, _C)])
    pltpu.sync_copy(work_v.at[pl.ds(0, _RPS - 4 * _C)],
                    acc.at[pl.ds(s * _RPS + 4 * _C, _RPS - 4 * _C)])

    stage_group(0, 0, issue=True)
    stage_group(0, 0, issue=False)
    stage_group(1, 1, issue=True)

    plsc.subcore_barrier()

    for b in range(_NBUF):
        pltpu.async_copy(src_hbm.at[col_v.at[0, b]], rows_v.at[b], gsem.at[b])

    @pl.loop(0, _NG, step=2)
    def _(g0):
        for gp in range(2):
            g = g0 + gp
            for jg in range(_G):
                b = jg % _NBUF
                pltpu.make_async_copy(src_hbm.at[col_v.at[gp, jg]],
                                      rows_v.at[b], gsem.at[b]).wait()

                @pl.loop(0, _C, step=16)
                def _(e0):
                    vv = val_v[gp, jg, pl.ds(e0, 16)]
                    for l in range(16):
                        v = vv[l]
                        for k in range(_D // 32):
                            w = rows_v[b, e0 + l, pl.ds(16 * k, 16)]
                            grp = plsc.bitcast(w, jnp.bfloat16)
                            lo, hi = plsc.unpack(
                                grp, format=plsc.PackFormat.INTERLEAVED)
                            work_v[e0 + l, pl.ds(32 * k, 16)] = lo * v
                            work_v[e0 + l, pl.ds(32 * k + 16, 16)] = hi * v

                pltpu.async_copy(work_v, acc.at[row_v.at[gp, jg]],
                                 ssem.at[b], add=True).wait()

                if jg == _G - _NBUF:
                    @pl.when(g + 1 < _NG)
                    def _():
                        stage_group(1 - gp, g + 1, issue=False)

                nxt = jg + _NBUF
                if nxt < _G:
                    pltpu.async_copy(src_hbm.at[col_v.at[gp, nxt]],
                                     rows_v.at[b], gsem.at[b])
                else:
                    @pl.when(g + 1 < _NG)
                    def _():
                        pltpu.async_copy(
                            src_hbm.at[col_v.at[1 - gp, nxt - _G]],
                            rows_v.at[b], gsem.at[b])

            @pl.when(g + 2 < _NG)
            def _():
                stage_group(gp, g + 2, issue=True)

    plsc.subcore_barrier()

    pltpu.sync_copy(acc.at[pl.ds(s * _RPS, _RPS)],
                    out_hbm.at[c, pl.ds(s * _RPS, _RPS)])


def _add_body(p_ref, o_ref):
    o_ref[...] = p_ref[0] + p_ref[1]


_BLK = 632


def _combine(parts):
    return pl.pallas_call(
        _add_body,
        grid=(_NPAD // _BLK,),
        in_specs=[pl.BlockSpec((_NC, _BLK, _D), lambda i: (0, i, 0))],
        out_specs=pl.BlockSpec((_BLK, _D), lambda i: (i, 0)),
        out_shape=jax.ShapeDtypeStruct((_NPAD, _D), jnp.float32),
    )(parts)


def _to_gather_fmt(x):
    v = x.reshape(_N, _D // 32, 2, 16)
    v = jnp.swapaxes(v, 2, 3)
    v = v.reshape(_N, _D // 2, 2).astype(jnp.bfloat16)
    return lax.bitcast_convert_type(v, jnp.int32)


def kernel(user_embed, item_embed, adj_indices, adj_values):
    x = jnp.concatenate([user_embed, item_embed], axis=0)
    pad = _EPAD - _E
    row = jnp.concatenate([adj_indices[0], jnp.zeros((pad,), jnp.int32)])
    col = jnp.concatenate([adj_indices[1], jnp.zeros((pad,), jnp.int32)])
    val = jnp.concatenate([adj_values, jnp.zeros((pad,), jnp.float32)])
    row = row.reshape(_NW, _NCH, _C)
    col = col.reshape(_NW, _NCH, _C)
    val = val.reshape(_NW, _NCH, _C)

    embs = [x]
    for _ in range(_HOPS):
        parts = _hop(_to_gather_fmt(x), col, row, val)
        x = _combine(parts)[: _N]
        embs.append(x)
    embs = jnp.stack(embs, axis=1)
    return embs[:_N_USERS], embs[_N_USERS:]


# speedup vs baseline: 1.7249x; 1.1127x over previous
import dataclasses
import functools

import jax
import jax.numpy as jnp
from jax import lax
from jax.experimental import pallas as pl
from jax.experimental.pallas import tpu as pltpu
from jax.experimental.pallas import tpu_sc as plsc

_N_USERS = 4000
_N_ITEMS = 6000
_N = _N_USERS + _N_ITEMS
_E = 320000
_D = 128
_HOPS = 3

_NC = 2
_NS = 16
_NW = _NC * _NS
_C = 128
_NBUF = 2
_G = 4
_NG = 20
_NCH = _G * _NG
_EPAD = _NW * _NCH * _C
_RPS = 632
_NPAD = _NS * _RPS

_mesh = plsc.VectorSubcoreMesh(core_axis_name="core", subcore_axis_name="subcore")

_cp = pltpu.CompilerParams(use_tc_tiling_on_sc=False)
if "needs_layout_passes" in pltpu.CompilerParams.__dataclass_fields__:
    _cp = dataclasses.replace(_cp, needs_layout_passes=False)


@functools.partial(
    pl.kernel,
    mesh=_mesh,
    compiler_params=_cp,
    out_type=jax.ShapeDtypeStruct((_NC, _NPAD, _D), jnp.float32),
    scratch_types=[
        pltpu.VMEM_SHARED((_NPAD, _D), jnp.float32),
        pltpu.VMEM((2, _G, _C), jnp.int32),
        pltpu.VMEM((2, 2 * _G, _C // 2), jnp.int32),
        pltpu.VMEM((2, _G, _C), jnp.float32),
        pltpu.VMEM((_NBUF, _C, _D // 2), jnp.int32),
        pltpu.VMEM((_C, _D), jnp.float32),
        pltpu.SemaphoreType.DMA((_NBUF,)),
        pltpu.SemaphoreType.DMA((2,)),
        pltpu.SemaphoreType.DMA((2,)),
    ],
)
def _hop(src_hbm, col_hbm, row_hbm, val_hbm, out_hbm,
         acc, col_v, row_v, val_v, rows_v, work_v, gsem, ssem, isem):
    c = lax.axis_index("core")
    s = lax.axis_index("subcore")
    wid = c * _NS + s

    def stage_group(slot, g, issue):
        copies = (
            pltpu.make_async_copy(col_hbm.at[wid, pl.ds(g * _G, _G)],
                                  col_v.at[slot], isem.at[slot]),
            pltpu.make_async_copy(row_hbm.at[wid, pl.ds(g * 2 * _G, 2 * _G)],
                                  row_v.at[slot], isem.at[slot]),
            pltpu.make_async_copy(val_hbm.at[wid, pl.ds(g * _G, _G)],
                                  val_v.at[slot], isem.at[slot]),
        )
        for cp in copies:
            if issue:
                cp.start()
            else:
                cp.wait()

    def scat_half(gp, jg, h):
        return pltpu.make_async_copy(
            work_v.at[pl.ds(h * (_C // 2), _C // 2)],
            acc.at[row_v.at[gp, 2 * jg + h]], ssem.at[h])

    @pl.loop(0, _C)
    def _(i):
        @pl.loop(0, _D, step=16)
        def _(j):
            work_v[i, pl.ds(j, 16)] = jnp.zeros((16,), jnp.float32)

    for k in range(4):
        pltpu.sync_copy(work_v, acc.at[pl.ds(s * _RPS + k * _C, _C)])
    pltpu.sync_copy(work_v.at[pl.ds(0, _RPS - 4 * _C)],
                    acc.at[pl.ds(s * _RPS + 4 * _C, _RPS - 4 * _C)])

    stage_group(0, 0, issue=True)
    stage_group(0, 0, issue=False)
    stage_group(1, 1, issue=True)

    plsc.subcore_barrier()

    for b in range(_NBUF):
        pltpu.async_copy(src_hbm.at[col_v.at[0, b]], rows_v.at[b], gsem.at[b])

    @pl.loop(0, _NG, step=2)
    def _(g0):
        for gp in range(2):
            g = g0 + gp
            for jg in range(_G):
                b = jg % _NBUF
                pltpu.make_async_copy(src_hbm.at[col_v.at[gp, jg]],
                                      rows_v.at[b], gsem.at[b]).wait()

                for h in range(2):
                    if jg == 0:
                        @pl.when(g > 0)
                        def _(h=h):
                            scat_half(gp, jg, h).wait()
                    else:
                        scat_half(gp, jg, h).wait()

                    @pl.loop(h * (_C // 2), (h + 1) * (_C // 2), step=16)
                    def _(e0):
                        vv = val_v[gp, jg, pl.ds(e0, 16)]
                        for l in range(16):
                            v = vv[l]
                            for k in range(_D // 32):
                                w = rows_v[b, e0 + l, pl.ds(16 * k, 16)]
                                grp = plsc.bitcast(w, jnp.bfloat16)
                                lo, hi = plsc.unpack(
                                    grp, format=plsc.PackFormat.INTERLEAVED)
                                work_v[e0 + l, pl.ds(32 * k, 16)] = lo * v
                                work_v[e0 + l, pl.ds(32 * k + 16, 16)] = hi * v

                    scat_half(gp, jg, h).start(add=True)

                if jg == _G - _NBUF:
                    @pl.when(g + 1 < _NG)
                    def _():
                        stage_group(1 - gp, g + 1, issue=False)

                nxt = jg + _NBUF
                if nxt < _G:
                    pltpu.async_copy(src_hbm.at[col_v.at[gp, nxt]],
                                     rows_v.at[b], gsem.at[b])
                else:
                    @pl.when(g + 1 < _NG)
                    def _():
                        pltpu.async_copy(
                            src_hbm.at[col_v.at[1 - gp, nxt - _G]],
                            rows_v.at[b], gsem.at[b])

            @pl.when(g + 2 < _NG)
            def _():
                stage_group(gp, g + 2, issue=True)

    for h in range(2):
        scat_half(1, _G - 1, h).wait()

    plsc.subcore_barrier()

    pltpu.sync_copy(acc.at[pl.ds(s * _RPS, _RPS)],
                    out_hbm.at[c, pl.ds(s * _RPS, _RPS)])


def _add_body(p_ref, o_ref):
    o_ref[...] = p_ref[0] + p_ref[1]


_BLK = 632


def _combine(parts):
    return pl.pallas_call(
        _add_body,
        grid=(_NPAD // _BLK,),
        in_specs=[pl.BlockSpec((_NC, _BLK, _D), lambda i: (0, i, 0))],
        out_specs=pl.BlockSpec((_BLK, _D), lambda i: (i, 0)),
        out_shape=jax.ShapeDtypeStruct((_NPAD, _D), jnp.float32),
    )(parts)


def _to_gather_fmt(x):
    v = x.reshape(_N, _D // 32, 2, 16)
    v = jnp.swapaxes(v, 2, 3)
    v = v.reshape(_N, _D // 2, 2).astype(jnp.bfloat16)
    return lax.bitcast_convert_type(v, jnp.int32)


def kernel(user_embed, item_embed, adj_indices, adj_values):
    x = jnp.concatenate([user_embed, item_embed], axis=0)
    pad = _EPAD - _E
    row = jnp.concatenate([adj_indices[0], jnp.zeros((pad,), jnp.int32)])
    col = jnp.concatenate([adj_indices[1], jnp.zeros((pad,), jnp.int32)])
    val = jnp.concatenate([adj_values, jnp.zeros((pad,), jnp.float32)])
    row = row.reshape(_NW, _NCH * 2, _C // 2)
    col = col.reshape(_NW, _NCH, _C)
    val = val.reshape(_NW, _NCH, _C)

    embs = [x]
    for _ in range(_HOPS):
        parts = _hop(_to_gather_fmt(x), col, row, val)
        x = _combine(parts)[: _N]
        embs.append(x)
    embs = jnp.stack(embs, axis=1)
    return embs[:_N_USERS], embs[_N_USERS:]


# speedup vs baseline: 2.0412x; 1.1834x over previous
import dataclasses
import functools

import jax
import jax.numpy as jnp
from jax import lax
from jax.experimental import pallas as pl
from jax.experimental.pallas import tpu as pltpu
from jax.experimental.pallas import tpu_sc as plsc

_N_USERS = 4000
_N_ITEMS = 6000
_N = _N_USERS + _N_ITEMS
_E = 320000
_D = 128
_HOPS = 3

_NC = 2
_NS = 16
_NW = _NC * _NS
_C = 128
_NBUF = 2
_G = 4
_NG = 20
_NCH = _G * _NG
_EPAD = _NW * _NCH * _C
_RPS = 632
_NPAD = _NS * _RPS

_mesh = plsc.VectorSubcoreMesh(core_axis_name="core", subcore_axis_name="subcore")

_cp = pltpu.CompilerParams(use_tc_tiling_on_sc=False)
if "needs_layout_passes" in pltpu.CompilerParams.__dataclass_fields__:
    _cp = dataclasses.replace(_cp, needs_layout_passes=False)


@functools.partial(
    pl.kernel,
    mesh=_mesh,
    compiler_params=_cp,
    out_type=jax.ShapeDtypeStruct((_NC, _NPAD, _D), jnp.float32),
    scratch_types=[
        pltpu.VMEM_SHARED((_NPAD, _D), jnp.float32),
        pltpu.VMEM((2, _G, _C), jnp.int32),
        pltpu.VMEM((2, 2 * _G, _C // 2), jnp.int32),
        pltpu.VMEM((2, _G, _C), jnp.float32),
        pltpu.VMEM((_NBUF, _C, _D // 2), jnp.int32),
        pltpu.VMEM((_C, _D), jnp.float32),
        pltpu.SemaphoreType.DMA((_NBUF,)),
        pltpu.SemaphoreType.DMA((2,)),
        pltpu.SemaphoreType.DMA((2,)),
    ],
)
def _hop(src_hbm, col_hbm, row_hbm, val_hbm, out_hbm,
         acc, col_v, row_v, val_v, rows_v, work_v, gsem, ssem, isem):
    c = lax.axis_index("core")
    s = lax.axis_index("subcore")
    wid = c * _NS + s

    def stage_group(slot, g, issue):
        copies = (
            pltpu.make_async_copy(col_hbm.at[wid, pl.ds(g * _G, _G)],
                                  col_v.at[slot], isem.at[slot]),
            pltpu.make_async_copy(row_hbm.at[wid, pl.ds(g * 2 * _G, 2 * _G)],
                                  row_v.at[slot], isem.at[slot]),
            pltpu.make_async_copy(val_hbm.at[wid, pl.ds(g * _G, _G)],
                                  val_v.at[slot], isem.at[slot]),
        )
        for cp in copies:
            if issue:
                cp.start()
            else:
                cp.wait()

    def scat_half(gp, jg, h):
        return pltpu.make_async_copy(
            work_v.at[pl.ds(h * (_C // 2), _C // 2)],
            acc.at[row_v.at[gp, 2 * jg + h]], ssem.at[h])

    @pl.loop(0, _C)
    def _(i):
        @pl.loop(0, _D, step=16)
        def _(j):
            work_v[i, pl.ds(j, 16)] = jnp.zeros((16,), jnp.float32)

    for k in range(4):
        pltpu.sync_copy(work_v, acc.at[pl.ds(s * _RPS + k * _C, _C)])
    pltpu.sync_copy(work_v.at[pl.ds(0, _RPS - 4 * _C)],
                    acc.at[pl.ds(s * _RPS + 4 * _C, _RPS - 4 * _C)])

    stage_group(0, 0, issue=True)
    stage_group(0, 0, issue=False)
    stage_group(1, 1, issue=True)

    plsc.subcore_barrier()

    for b in range(_NBUF):
        pltpu.async_copy(src_hbm.at[col_v.at[0, b]], rows_v.at[b], gsem.at[b])

    @pl.loop(0, _NG, step=2)
    def _(g0):
        for gp in range(2):
            g = g0 + gp
            for jg in range(_G):
                b = jg % _NBUF
                pltpu.make_async_copy(src_hbm.at[col_v.at[gp, jg]],
                                      rows_v.at[b], gsem.at[b]).wait()

                for h in range(2):
                    if jg == 0:
                        @pl.when(g > 0)
                        def _(h=h):
                            scat_half(gp, jg, h).wait()
                    else:
                        scat_half(gp, jg, h).wait()

                    gp_ix = jnp.full((16,), gp, jnp.int32)
                    jg_ix = jnp.full((16,), jg, jnp.int32)

                    @plsc.parallel_loop(h * (_C // 2), (h + 1) * (_C // 2),
                                        1, unroll=4)
                    def _(e):
                        e_ix = jax.lax.broadcast_in_dim(e, (16,), ())
                        vsp = plsc.load_gather(val_v, [gp_ix, jg_ix, e_ix])
                        for k in range(_D // 32):
                            w = rows_v[b, e, pl.ds(16 * k, 16)]
                            grp = plsc.bitcast(w, jnp.bfloat16)
                            lo, hi = plsc.unpack(
                                grp, format=plsc.PackFormat.INTERLEAVED)
                            work_v[e, pl.ds(32 * k, 16)] = lo * vsp
                            work_v[e, pl.ds(32 * k + 16, 16)] = hi * vsp

                    scat_half(gp, jg, h).start(add=True)

                if jg == _G - _NBUF:
                    @pl.when(g + 1 < _NG)
                    def _():
                        stage_group(1 - gp, g + 1, issue=False)

                nxt = jg + _NBUF
                if nxt < _G:
                    pltpu.async_copy(src_hbm.at[col_v.at[gp, nxt]],
                                     rows_v.at[b], gsem.at[b])
                else:
                    @pl.when(g + 1 < _NG)
                    def _():
                        pltpu.async_copy(
                            src_hbm.at[col_v.at[1 - gp, nxt - _G]],
                            rows_v.at[b], gsem.at[b])

            @pl.when(g + 2 < _NG)
            def _():
                stage_group(gp, g + 2, issue=True)

    for h in range(2):
        scat_half(1, _G - 1, h).wait()

    plsc.subcore_barrier()

    pltpu.sync_copy(acc.at[pl.ds(s * _RPS, _RPS)],
                    out_hbm.at[c, pl.ds(s * _RPS, _RPS)])


def _add_body(p_ref, o_ref):
    o_ref[...] = p_ref[0] + p_ref[1]


_BLK = 632


def _combine(parts):
    return pl.pallas_call(
        _add_body,
        grid=(_NPAD // _BLK,),
        in_specs=[pl.BlockSpec((_NC, _BLK, _D), lambda i: (0, i, 0))],
        out_specs=pl.BlockSpec((_BLK, _D), lambda i: (i, 0)),
        out_shape=jax.ShapeDtypeStruct((_NPAD, _D), jnp.float32),
    )(parts)


def _to_gather_fmt(x):
    v = x.reshape(_N, _D // 32, 2, 16)
    v = jnp.swapaxes(v, 2, 3)
    v = v.reshape(_N, _D // 2, 2).astype(jnp.bfloat16)
    return lax.bitcast_convert_type(v, jnp.int32)


def kernel(user_embed, item_embed, adj_indices, adj_values):
    x = jnp.concatenate([user_embed, item_embed], axis=0)
    pad = _EPAD - _E
    row = jnp.concatenate([adj_indices[0], jnp.zeros((pad,), jnp.int32)])
    col = jnp.concatenate([adj_indices[1], jnp.zeros((pad,), jnp.int32)])
    val = jnp.concatenate([adj_values, jnp.zeros((pad,), jnp.float32)])
    row = row.reshape(_NW, _NCH * 2, _C // 2)
    col = col.reshape(_NW, _NCH, _C)
    val = val.reshape(_NW, _NCH, _C)

    embs = [x]
    for _ in range(_HOPS):
        parts = _hop(_to_gather_fmt(x), col, row, val)
        x = _combine(parts)[: _N]
        embs.append(x)
    embs = jnp.stack(embs, axis=1)
    return embs[:_N_USERS], embs[_N_USERS:]


# speedup vs baseline: 2.8760x; 1.4090x over previous
import dataclasses
import functools

import jax
import jax.numpy as jnp
from jax import lax
from jax.experimental import pallas as pl
from jax.experimental.pallas import tpu as pltpu
from jax.experimental.pallas import tpu_sc as plsc

_N_USERS = 4000
_N_ITEMS = 6000
_N = _N_USERS + _N_ITEMS
_E = 320000
_D = 128
_DH = _D // 2
_HOPS = 3

_NC = 2
_NS = 16
_NW = _NC * _NS
_C = 128
_NBUF = 4
_G = 4
_NG = 20
_NCH = _G * _NG
_EPAD = _NW * _NCH * _C
_RPS = _N // _NS

_mesh = plsc.VectorSubcoreMesh(core_axis_name="core", subcore_axis_name="subcore")

_cp = pltpu.CompilerParams(use_tc_tiling_on_sc=False)
if "needs_layout_passes" in pltpu.CompilerParams.__dataclass_fields__:
    _cp = dataclasses.replace(_cp, needs_layout_passes=False)


@functools.partial(
    pl.kernel,
    mesh=_mesh,
    compiler_params=_cp,
    out_type=jax.ShapeDtypeStruct((_NC, _N, _D), jnp.float32),
    scratch_types=[
        pltpu.VMEM_SHARED((_N, _DH), jnp.float32),
        pltpu.VMEM_SHARED((_N, _DH), jnp.float32),
        pltpu.VMEM((2, _G, _C), jnp.int32),
        pltpu.VMEM((_NCH, _C), jnp.int32),
        pltpu.VMEM((2, _G, _C), jnp.float32),
        pltpu.VMEM((_NBUF, _C, _DH), jnp.float32),
        pltpu.SemaphoreType.DMA((_NBUF,)),
        pltpu.SemaphoreType.DMA((_NBUF,)),
        pltpu.SemaphoreType.DMA((2,)),
    ],
)
def _hop(src_hbm, col_hbm, row_hbm, val_hbm, out_hbm,
         src_s, acc, col_v, row_v, val_v, rows_v, gsem, ssem, isem):
    c = lax.axis_index("core")
    s = lax.axis_index("subcore")
    wid = c * _NS + s

    def stage_group(slot, g, issue):
        copies = (
            pltpu.make_async_copy(col_hbm.at[wid, pl.ds(g * _G, _G)],
                                  col_v.at[slot], isem.at[slot]),
            pltpu.make_async_copy(val_hbm.at[wid, pl.ds(g * _G, _G)],
                                  val_v.at[slot], isem.at[slot]),
        )
        for cp in copies:
            if issue:
                cp.start()
            else:
                cp.wait()

    def gather(slot_gp, jg, buf):
        return pltpu.make_async_copy(src_s.at[col_v.at[slot_gp, jg]],
                                     rows_v.at[buf], gsem.at[buf])

    def scat(j, buf):
        return pltpu.make_async_copy(rows_v.at[buf], acc.at[row_v.at[j]],
                                     ssem.at[buf])

    for h in range(2):
        pltpu.sync_copy(src_hbm.at[h, pl.ds(s * _RPS, _RPS)],
                        src_s.at[pl.ds(s * _RPS, _RPS)])

        @pl.loop(0, _C)
        def _(i):
            @pl.loop(0, _DH, step=16)
            def _(j):
                rows_v[0, i, pl.ds(j, 16)] = jnp.zeros((16,), jnp.float32)

        for k in range(4):
            pltpu.sync_copy(rows_v.at[0],
                            acc.at[pl.ds(s * _RPS + k * _C, _C)])
        pltpu.sync_copy(rows_v.at[0, pl.ds(0, _RPS - 4 * _C)],
                        acc.at[pl.ds(s * _RPS + 4 * _C, _RPS - 4 * _C)])

        pltpu.sync_copy(row_hbm.at[wid], row_v)
        stage_group(0, 0, issue=True)
        stage_group(0, 0, issue=False)
        stage_group(1, 1, issue=True)

        plsc.subcore_barrier()

        gather(0, 0, 0).start()
        gather(0, 1, 1).start()

        @pl.loop(0, _NG, step=2)
        def _(g0):
            for gp in range(2):
                g = g0 + gp
                j_base = g * _G
                for jg in range(_G):
                    b = jg
                    b2 = (jg + 2) % _NBUF

                    if jg < 2:
                        @pl.when(g > 0)
                        def _(jg=jg, b2=b2):
                            scat(j_base + jg - 2, b2).wait()
                    else:
                        scat(j_base + jg - 2, b2).wait()

                    if jg == _G - 2:
                        @pl.when(g + 1 < _NG)
                        def _():
                            stage_group(1 - gp, g + 1, issue=False)

                    if jg < 2:
                        gather(gp, jg + 2, b2).start()
                    else:
                        @pl.when(g + 1 < _NG)
                        def _(jg=jg, b2=b2, gp=gp):
                            gather(1 - gp, jg - 2, b2).start()

                    gather(gp, jg, b).wait()

                    gp_ix = jnp.full((16,), gp, jnp.int32)
                    jg_ix = jnp.full((16,), jg, jnp.int32)

                    @plsc.parallel_loop(0, _C, 1, unroll=8)
                    def _(e, b=b, gp_ix=gp_ix, jg_ix=jg_ix):
                        e_ix = lax.broadcast_in_dim(e, (16,), ())
                        vsp = plsc.load_gather(val_v, [gp_ix, jg_ix, e_ix])
                        for k in range(_DH // 16):
                            sl = pl.ds(16 * k, 16)
                            rows_v[b, e, sl] = rows_v[b, e, sl] * vsp

                    scat(j_base + jg, b).start(add=True)

                @pl.when(g + 2 < _NG)
                def _(gp=gp, g=g):
                    stage_group(gp, g + 2, issue=True)

        scat(_NCH - 2, 2).wait()
        scat(_NCH - 1, 3).wait()

        plsc.subcore_barrier()

        pltpu.sync_copy(acc.at[pl.ds(s * _RPS, _RPS)],
                        out_hbm.at[c, pl.ds(s * _RPS, _RPS),
                                   pl.ds(h * _DH, _DH)])


def _add_body(p_ref, o_ref, h_ref):
    x = p_ref[0] + p_ref[1]
    o_ref[...] = x
    h_ref[0] = x[:, :_DH]
    h_ref[1] = x[:, _DH:]


_BLK = 1000


def _combine(parts):
    return pl.pallas_call(
        _add_body,
        grid=(_N // _BLK,),
        in_specs=[pl.BlockSpec((_NC, _BLK, _D), lambda i: (0, i, 0))],
        out_specs=[pl.BlockSpec((_BLK, _D), lambda i: (i, 0)),
                   pl.BlockSpec((2, _BLK, _DH), lambda i: (0, i, 0))],
        out_shape=[jax.ShapeDtypeStruct((_N, _D), jnp.float32),
                   jax.ShapeDtypeStruct((2, _N, _DH), jnp.float32)],
    )(parts)


def kernel(user_embed, item_embed, adj_indices, adj_values):
    x = jnp.concatenate([user_embed, item_embed], axis=0)
    halves = jnp.stack([x[:, :_DH], x[:, _DH:]])
    pad = _EPAD - _E
    row = jnp.concatenate([adj_indices[0], jnp.zeros((pad,), jnp.int32)])
    col = jnp.concatenate([adj_indices[1], jnp.zeros((pad,), jnp.int32)])
    val = jnp.concatenate([adj_values, jnp.zeros((pad,), jnp.float32)])
    row = row.reshape(_NW, _NCH, _C)
    col = col.reshape(_NW, _NCH, _C)
    val = val.reshape(_NW, _NCH, _C)

    embs = [x]
    for _ in range(_HOPS):
        parts = _hop(halves, col, row, val)
        x, halves = _combine(parts)
        embs.append(x)
    embs = jnp.stack(embs, axis=1)
    return embs[:_N_USERS], embs[_N_USERS:]


# speedup vs baseline: 2.9256x; 1.0172x over previous
import dataclasses
import functools

import jax
import jax.numpy as jnp
from jax import lax
from jax.experimental import pallas as pl
from jax.experimental.pallas import tpu as pltpu
from jax.experimental.pallas import tpu_sc as plsc

_N_USERS = 4000
_N_ITEMS = 6000
_N = _N_USERS + _N_ITEMS
_E = 320000
_D = 128
_DH = _D // 2
_HOPS = 3

_NC = 2
_NS = 16
_NW = _NC * _NS
_C = 128
_NBUF = 4
_G = 4
_NG = 20
_NCH = _G * _NG
_EPAD = _NW * _NCH * _C
_RPS = _N // _NS

_mesh = plsc.VectorSubcoreMesh(core_axis_name="core", subcore_axis_name="subcore")

_cp = pltpu.CompilerParams(use_tc_tiling_on_sc=False)
if "needs_layout_passes" in pltpu.CompilerParams.__dataclass_fields__:
    _cp = dataclasses.replace(_cp, needs_layout_passes=False)


@functools.partial(
    pl.kernel,
    mesh=_mesh,
    compiler_params=_cp,
    out_type=jax.ShapeDtypeStruct((_NC, _N, _D), jnp.float32),
    scratch_types=[
        pltpu.VMEM_SHARED((_N, _DH), jnp.float32),
        pltpu.VMEM_SHARED((_N, _DH), jnp.float32),
        pltpu.VMEM((2, _G, _C), jnp.int32),
        pltpu.VMEM((_NCH, _C), jnp.int32),
        pltpu.VMEM((2, _G, _C), jnp.float32),
        pltpu.VMEM((_NBUF, _C, _DH), jnp.float32),
        pltpu.SemaphoreType.DMA((_NBUF,)),
        pltpu.SemaphoreType.DMA((_NBUF,)),
        pltpu.SemaphoreType.DMA((2,)),
    ],
)
def _hop(src_hbm, col_hbm, row_hbm, val_hbm, out_hbm,
         src_s, acc, col_v, row_v, val_v, rows_v, gsem, ssem, isem):
    c = lax.axis_index("core")
    s = lax.axis_index("subcore")
    wid = c * _NS + s

    def stage_group(slot, g, issue):
        copies = (
            pltpu.make_async_copy(col_hbm.at[wid, pl.ds(g * _G, _G)],
                                  col_v.at[slot], isem.at[slot]),
            pltpu.make_async_copy(val_hbm.at[wid, pl.ds(g * _G, _G)],
                                  val_v.at[slot], isem.at[slot]),
        )
        for cp in copies:
            if issue:
                cp.start()
            else:
                cp.wait()

    def gather(slot_gp, jg, buf):
        return pltpu.make_async_copy(src_s.at[col_v.at[slot_gp, jg]],
                                     rows_v.at[buf], gsem.at[buf])

    def scat(j, buf):
        return pltpu.make_async_copy(rows_v.at[buf], acc.at[row_v.at[j]],
                                     ssem.at[buf])

    for h in range(2):
        pltpu.sync_copy(src_hbm.at[h, pl.ds(s * _RPS, _RPS)],
                        src_s.at[pl.ds(s * _RPS, _RPS)])

        @pl.loop(0, _C)
        def _(i):
            @pl.loop(0, _DH, step=16)
            def _(j):
                rows_v[0, i, pl.ds(j, 16)] = jnp.zeros((16,), jnp.float32)

        for k in range(4):
            pltpu.sync_copy(rows_v.at[0],
                            acc.at[pl.ds(s * _RPS + k * _C, _C)])
        pltpu.sync_copy(rows_v.at[0, pl.ds(0, _RPS - 4 * _C)],
                        acc.at[pl.ds(s * _RPS + 4 * _C, _RPS - 4 * _C)])

        pltpu.sync_copy(row_hbm.at[wid], row_v)
        stage_group(0, 0, issue=True)
        stage_group(0, 0, issue=False)
        stage_group(1, 1, issue=True)

        plsc.subcore_barrier()

        gather(0, 0, 0).start()
        gather(0, 1, 1).start()

        @pl.loop(0, _NG, step=2)
        def _(g0):
            for gp in range(2):
                g = g0 + gp
                j_base = g * _G
                for jg in range(_G):
                    b = jg
                    b2 = (jg + 2) % _NBUF

                    if jg < 2:
                        @pl.when(g > 0)
                        def _(jg=jg, b2=b2):
                            scat(j_base + jg - 2, b2).wait()
                    else:
                        scat(j_base + jg - 2, b2).wait()

                    if jg == _G - 2:
                        @pl.when(g + 1 < _NG)
                        def _():
                            stage_group(1 - gp, g + 1, issue=False)

                    if jg < 2:
                        gather(gp, jg + 2, b2).start()
                    else:
                        @pl.when(g + 1 < _NG)
                        def _(jg=jg, b2=b2, gp=gp):
                            gather(1 - gp, jg - 2, b2).start()

                    gather(gp, jg, b).wait()

                    gp_ix = jnp.full((16,), gp, jnp.int32)
                    jg_ix = jnp.full((16,), jg, jnp.int32)

                    @plsc.parallel_loop(0, _C, 1, unroll=8)
                    def _(e, b=b, gp_ix=gp_ix, jg_ix=jg_ix):
                        e_ix = lax.broadcast_in_dim(e, (16,), ())
                        vsp = plsc.load_gather(val_v, [gp_ix, jg_ix, e_ix])
                        for k in range(_DH // 16):
                            sl = pl.ds(16 * k, 16)
                            rows_v[b, e, sl] = rows_v[b, e, sl] * vsp

                    scat(j_base + jg, b).start(add=True)

                @pl.when(g + 2 < _NG)
                def _(gp=gp, g=g):
                    stage_group(gp, g + 2, issue=True)

        scat(_NCH - 2, (_NCH - 2) % _NBUF).wait()
        scat(_NCH - 1, (_NCH - 1) % _NBUF).wait()

        plsc.subcore_barrier()

        pltpu.sync_copy(acc.at[pl.ds(s * _RPS, _RPS)],
                        out_hbm.at[c, pl.ds(s * _RPS, _RPS),
                                   pl.ds(h * _DH, _DH)])


def _add_body(p_ref, o_ref, h_ref):
    x = p_ref[0] + p_ref[1]
    o_ref[...] = x
    h_ref[0] = x[:, :_DH]
    h_ref[1] = x[:, _DH:]


_BLK = 1000


def _combine(parts):
    return pl.pallas_call(
        _add_body,
        grid=(_N // _BLK,),
        in_specs=[pl.BlockSpec((_NC, _BLK, _D), lambda i: (0, i, 0))],
        out_specs=[pl.BlockSpec((_BLK, _D), lambda i: (i, 0)),
                   pl.BlockSpec((2, _BLK, _DH), lambda i: (0, i, 0))],
        out_shape=[jax.ShapeDtypeStruct((_N, _D), jnp.float32),
                   jax.ShapeDtypeStruct((2, _N, _DH), jnp.float32)],
    )(parts)


def _asm_body(x0_ref, x1_ref, x2_ref, x3_ref, o_ref):
    o_ref[:, 0, :] = x0_ref[...]
    o_ref[:, 1, :] = x1_ref[...]
    o_ref[:, 2, :] = x2_ref[...]
    o_ref[:, 3, :] = x3_ref[...]


def _assemble(embs, base, rows):
    blk = 1000
    off = base // blk
    return pl.pallas_call(
        _asm_body,
        grid=(rows // blk,),
        in_specs=[pl.BlockSpec((blk, _D), lambda i, o=off: (i + o, 0))
                  for _ in range(_HOPS + 1)],
        out_specs=pl.BlockSpec((blk, _HOPS + 1, _D), lambda i: (i, 0, 0)),
        out_shape=jax.ShapeDtypeStruct((rows, _HOPS + 1, _D), jnp.float32),
    )(*embs)


def kernel(user_embed, item_embed, adj_indices, adj_values):
    x = jnp.concatenate([user_embed, item_embed], axis=0)
    halves = jnp.stack([x[:, :_DH], x[:, _DH:]])
    pad = _EPAD - _E
    row = jnp.concatenate([adj_indices[0], jnp.zeros((pad,), jnp.int32)])
    col = jnp.concatenate([adj_indices[1], jnp.zeros((pad,), jnp.int32)])
    val = jnp.concatenate([adj_values, jnp.zeros((pad,), jnp.float32)])
    row = row.reshape(_NW, _NCH, _C)
    col = col.reshape(_NW, _NCH, _C)
    val = val.reshape(_NW, _NCH, _C)

    embs = [x]
    for _ in range(_HOPS):
        parts = _hop(halves, col, row, val)
        x, halves = _combine(parts)
        embs.append(x)
    return (_assemble(embs, 0, _N_USERS),
            _assemble(embs, _N_USERS, _N_ITEMS))


# speedup vs baseline: 3.1231x; 1.0675x over previous
import dataclasses
import functools

import jax
import jax.numpy as jnp
from jax import lax
from jax.experimental import pallas as pl
from jax.experimental.pallas import tpu as pltpu
from jax.experimental.pallas import tpu_sc as plsc

_N_USERS = 4000
_N_ITEMS = 6000
_N = _N_USERS + _N_ITEMS
_E = 320000
_D = 128
_DH = _D // 2
_HOPS = 3

_NC = 2
_NS = 16
_NW = _NC * _NS
_C = 128
_NBUF = 4
_G = 4
_NG = 20
_NCH = _G * _NG
_EPAD = _NW * _NCH * _C
_RPS = _N // _NS

_mesh = plsc.VectorSubcoreMesh(core_axis_name="core", subcore_axis_name="subcore")

_cp = pltpu.CompilerParams(use_tc_tiling_on_sc=False)
if "needs_layout_passes" in pltpu.CompilerParams.__dataclass_fields__:
    _cp = dataclasses.replace(_cp, needs_layout_passes=False)


@functools.partial(
    pl.kernel,
    mesh=_mesh,
    compiler_params=_cp,
    out_type=jax.ShapeDtypeStruct((_NC, _N, _D), jnp.float32),
    scratch_types=[
        pltpu.VMEM_SHARED((_N, _DH), jnp.float32),
        pltpu.VMEM_SHARED((_N, _DH), jnp.float32),
        pltpu.VMEM((2, _G, _C), jnp.int32),
        pltpu.VMEM((_NCH, _C), jnp.int32),
        pltpu.VMEM((2, _G, _C), jnp.float32),
        pltpu.VMEM((_NBUF, _C, _DH), jnp.float32),
        pltpu.SemaphoreType.DMA((_NBUF,)),
        pltpu.SemaphoreType.DMA((_NBUF,)),
        pltpu.SemaphoreType.DMA((2,)),
    ],
)
def _hop(src_hbm, col_hbm, row_hbm, val_hbm, out_hbm,
         src_s, acc, col_v, row_v, val_v, rows_v, gsem, ssem, isem):
    c = lax.axis_index("core")
    s = lax.axis_index("subcore")
    wid = c * _NS + s

    def stage_group(slot, g, issue):
        copies = (
            pltpu.make_async_copy(col_hbm.at[wid, pl.ds(g * _G, _G)],
                                  col_v.at[slot], isem.at[slot]),
            pltpu.make_async_copy(val_hbm.at[wid, pl.ds(g * _G, _G)],
                                  val_v.at[slot], isem.at[slot]),
        )
        for cp in copies:
            if issue:
                cp.start()
            else:
                cp.wait()

    def gather(slot_gp, jg, buf):
        return pltpu.make_async_copy(src_s.at[col_v.at[slot_gp, jg]],
                                     rows_v.at[buf], gsem.at[buf])

    def scat(j, buf):
        return pltpu.make_async_copy(rows_v.at[buf], acc.at[row_v.at[j]],
                                     ssem.at[buf])

    for h in range(2):
        pltpu.sync_copy(src_hbm.at[pl.ds(s * _RPS, _RPS), pl.ds(h * _DH, _DH)],
                        src_s.at[pl.ds(s * _RPS, _RPS)])

        @pl.loop(0, _C)
        def _(i):
            @pl.loop(0, _DH, step=16)
            def _(j):
                rows_v[0, i, pl.ds(j, 16)] = jnp.zeros((16,), jnp.float32)

        for k in range(4):
            pltpu.sync_copy(rows_v.at[0],
                            acc.at[pl.ds(s * _RPS + k * _C, _C)])
        pltpu.sync_copy(rows_v.at[0, pl.ds(0, _RPS - 4 * _C)],
                        acc.at[pl.ds(s * _RPS + 4 * _C, _RPS - 4 * _C)])

        pltpu.sync_copy(row_hbm.at[wid], row_v)
        stage_group(0, 0, issue=True)
        stage_group(0, 0, issue=False)
        stage_group(1, 1, issue=True)

        plsc.subcore_barrier()

        gather(0, 0, 0).start()
        gather(0, 1, 1).start()

        @pl.loop(0, _NG, step=2)
        def _(g0):
            for gp in range(2):
                g = g0 + gp
                j_base = g * _G
                for jg in range(_G):
                    b = jg
                    b2 = (jg + 2) % _NBUF

                    if jg < 2:
                        @pl.when(g > 0)
                        def _(jg=jg, b2=b2):
                            scat(j_base + jg - 2, b2).wait()
                    else:
                        scat(j_base + jg - 2, b2).wait()

                    if jg == _G - 2:
                        @pl.when(g + 1 < _NG)
                        def _():
                            stage_group(1 - gp, g + 1, issue=False)

                    if jg < 2:
                        gather(gp, jg + 2, b2).start()
                    else:
                        @pl.when(g + 1 < _NG)
                        def _(jg=jg, b2=b2, gp=gp):
                            gather(1 - gp, jg - 2, b2).start()

                    gather(gp, jg, b).wait()

                    gp_ix = jnp.full((16,), gp, jnp.int32)
                    jg_ix = jnp.full((16,), jg, jnp.int32)

                    @plsc.parallel_loop(0, _C, 1, unroll=8)
                    def _(e, b=b, gp_ix=gp_ix, jg_ix=jg_ix):
                        e_ix = lax.broadcast_in_dim(e, (16,), ())
                        vsp = plsc.load_gather(val_v, [gp_ix, jg_ix, e_ix])
                        for k in range(_DH // 16):
                            sl = pl.ds(16 * k, 16)
                            rows_v[b, e, sl] = rows_v[b, e, sl] * vsp

                    scat(j_base + jg, b).start(add=True)

                @pl.when(g + 2 < _NG)
                def _(gp=gp, g=g):
                    stage_group(gp, g + 2, issue=True)

        scat(_NCH - 2, (_NCH - 2) % _NBUF).wait()
        scat(_NCH - 1, (_NCH - 1) % _NBUF).wait()

        plsc.subcore_barrier()

        pltpu.sync_copy(acc.at[pl.ds(s * _RPS, _RPS)],
                        out_hbm.at[c, pl.ds(s * _RPS, _RPS),
                                   pl.ds(h * _DH, _DH)])


def _add_body(p_ref, o_ref):
    o_ref[...] = p_ref[0] + p_ref[1]


_BLK = 1000


def _combine(parts):
    return pl.pallas_call(
        _add_body,
        grid=(_N // _BLK,),
        in_specs=[pl.BlockSpec((_NC, _BLK, _D), lambda i: (0, i, 0))],
        out_specs=pl.BlockSpec((_BLK, _D), lambda i: (i, 0)),
        out_shape=jax.ShapeDtypeStruct((_N, _D), jnp.float32),
    )(parts)


def _asm_body(x0_ref, x1_ref, x2_ref, x3_ref, o_ref):
    o_ref[:, 0, :] = x0_ref[...]
    o_ref[:, 1, :] = x1_ref[...]
    o_ref[:, 2, :] = x2_ref[...]
    o_ref[:, 3, :] = x3_ref[...]


def _assemble(embs, base, rows):
    blk = 1000
    off = base // blk
    return pl.pallas_call(
        _asm_body,
        grid=(rows // blk,),
        in_specs=[pl.BlockSpec((blk, _D), lambda i, o=off: (i + o, 0))
                  for _ in range(_HOPS + 1)],
        out_specs=pl.BlockSpec((blk, _HOPS + 1, _D), lambda i: (i, 0, 0)),
        out_shape=jax.ShapeDtypeStruct((rows, _HOPS + 1, _D), jnp.float32),
    )(*embs)


def kernel(user_embed, item_embed, adj_indices, adj_values):
    x = jnp.concatenate([user_embed, item_embed], axis=0)
    pad = _EPAD - _E
    row = jnp.concatenate([adj_indices[0], jnp.zeros((pad,), jnp.int32)])
    col = jnp.concatenate([adj_indices[1], jnp.zeros((pad,), jnp.int32)])
    val = jnp.concatenate([adj_values, jnp.zeros((pad,), jnp.float32)])
    row = row.reshape(_NW, _NCH, _C)
    col = col.reshape(_NW, _NCH, _C)
    val = val.reshape(_NW, _NCH, _C)

    embs = [x]
    for _ in range(_HOPS):
        parts = _hop(x, col, row, val)
        x = _combine(parts)
        embs.append(x)
    return (_assemble(embs, 0, _N_USERS),
            _assemble(embs, _N_USERS, _N_ITEMS))


# speedup vs baseline: 3.1772x; 1.0173x over previous
import dataclasses
import functools

import jax
import jax.numpy as jnp
from jax import lax
from jax.experimental import pallas as pl
from jax.experimental.pallas import tpu as pltpu
from jax.experimental.pallas import tpu_sc as plsc

_N_USERS = 4000
_N_ITEMS = 6000
_N = _N_USERS + _N_ITEMS
_E = 320000
_D = 128
_DH = _D // 2
_HOPS = 3

_NC = 2
_NS = 16
_NW = _NC * _NS
_C = 128
_NBUF = 4
_G = 4
_NG = 20
_NCH = _G * _NG
_EPAD = _NW * _NCH * _C
_RPS = _N // _NS

_mesh = plsc.VectorSubcoreMesh(core_axis_name="core", subcore_axis_name="subcore")

_cp = pltpu.CompilerParams(use_tc_tiling_on_sc=False)
if "needs_layout_passes" in pltpu.CompilerParams.__dataclass_fields__:
    _cp = dataclasses.replace(_cp, needs_layout_passes=False)


@functools.partial(
    pl.kernel,
    mesh=_mesh,
    compiler_params=_cp,
    out_type=jax.ShapeDtypeStruct((_NC, _N, _D), jnp.float32),
    scratch_types=[
        pltpu.VMEM_SHARED((_N, _DH), jnp.float32),
        pltpu.VMEM_SHARED((_N, _DH), jnp.float32),
        pltpu.VMEM((2, _G, _C), jnp.int32),
        pltpu.VMEM((_NCH, _C), jnp.int32),
        pltpu.VMEM((2, _G, _C), jnp.float32),
        pltpu.VMEM((_NBUF, _C, _DH), jnp.float32),
        pltpu.SemaphoreType.DMA((_NBUF,)),
        pltpu.SemaphoreType.DMA((_NBUF,)),
        pltpu.SemaphoreType.DMA((2,)),
    ],
)
def _hop(src_hbm, col_hbm, row_hbm, val_hbm, out_hbm,
         src_s, acc, col_v, row_v, val_v, rows_v, gsem, ssem, isem):
    c = lax.axis_index("core")
    s = lax.axis_index("subcore")
    wid = c * _NS + s

    def stage_group(slot, g, issue):
        copies = (
            pltpu.make_async_copy(col_hbm.at[wid, pl.ds(g * _G, _G)],
                                  col_v.at[slot], isem.at[slot]),
            pltpu.make_async_copy(val_hbm.at[wid, pl.ds(g * _G, _G)],
                                  val_v.at[slot], isem.at[slot]),
        )
        for cp in copies:
            if issue:
                cp.start()
            else:
                cp.wait()

    def gather(slot_gp, jg, buf):
        return pltpu.make_async_copy(src_s.at[col_v.at[slot_gp, jg]],
                                     rows_v.at[buf], gsem.at[buf])

    def scat(j, buf):
        return pltpu.make_async_copy(rows_v.at[buf], acc.at[row_v.at[j]],
                                     ssem.at[buf])

    for h in range(2):
        pltpu.sync_copy(src_hbm.at[pl.ds(s * _RPS, _RPS), pl.ds(h * _DH, _DH)],
                        src_s.at[pl.ds(s * _RPS, _RPS)])

        @pl.loop(0, _C)
        def _(i):
            @pl.loop(0, _DH, step=16)
            def _(j):
                rows_v[0, i, pl.ds(j, 16)] = jnp.zeros((16,), jnp.float32)

        for k in range(4):
            pltpu.sync_copy(rows_v.at[0],
                            acc.at[pl.ds(s * _RPS + k * _C, _C)])
        pltpu.sync_copy(rows_v.at[0, pl.ds(0, _RPS - 4 * _C)],
                        acc.at[pl.ds(s * _RPS + 4 * _C, _RPS - 4 * _C)])

        pltpu.sync_copy(row_hbm.at[wid], row_v)
        stage_group(0, 0, issue=True)
        stage_group(0, 0, issue=False)
        stage_group(1, 1, issue=True)

        plsc.subcore_barrier()

        gather(0, 0, 0).start()
        gather(0, 1, 1).start()

        @pl.loop(0, _NG, step=2)
        def _(g0):
            for gp in range(2):
                g = g0 + gp
                j_base = g * _G
                for jg in range(_G):
                    b = jg
                    b2 = (jg + 2) % _NBUF

                    if jg < 2:
                        @pl.when(g > 0)
                        def _(jg=jg, b2=b2):
                            scat(j_base + jg - 2, b2).wait()
                    else:
                        scat(j_base + jg - 2, b2).wait()

                    if jg == _G - 2:
                        @pl.when(g + 1 < _NG)
                        def _():
                            stage_group(1 - gp, g + 1, issue=False)

                    if jg < 2:
                        gather(gp, jg + 2, b2).start()
                    else:
                        @pl.when(g + 1 < _NG)
                        def _(jg=jg, b2=b2, gp=gp):
                            gather(1 - gp, jg - 2, b2).start()

                    gather(gp, jg, b).wait()

                    gp_ix = jnp.full((16,), gp, jnp.int32)
                    jg_ix = jnp.full((16,), jg, jnp.int32)

                    @plsc.parallel_loop(0, _C, 1, unroll=8)
                    def _(e, b=b, gp_ix=gp_ix, jg_ix=jg_ix):
                        e_ix = lax.broadcast_in_dim(e, (16,), ())
                        vsp = plsc.load_gather(val_v, [gp_ix, jg_ix, e_ix])
                        for k in range(_DH // 16):
                            sl = pl.ds(16 * k, 16)
                            rows_v[b, e, sl] = rows_v[b, e, sl] * vsp

                    scat(j_base + jg, b).start(add=True)

                @pl.when(g + 2 < _NG)
                def _(gp=gp, g=g):
                    stage_group(gp, g + 2, issue=True)

        scat(_NCH - 2, (_NCH - 2) % _NBUF).wait()
        scat(_NCH - 1, (_NCH - 1) % _NBUF).wait()

        plsc.subcore_barrier()

        pltpu.sync_copy(acc.at[pl.ds(s * _RPS, _RPS)],
                        out_hbm.at[c, pl.ds(s * _RPS, _RPS),
                                   pl.ds(h * _DH, _DH)])


def _add_body(p_ref, o_ref):
    o_ref[...] = p_ref[0] + p_ref[1]


_BLK = 1000


def _combine(parts):
    return pl.pallas_call(
        _add_body,
        grid=(_N // _BLK,),
        in_specs=[pl.BlockSpec((_NC, _BLK, _D), lambda i: (0, i, 0))],
        out_specs=pl.BlockSpec((_BLK, _D), lambda i: (i, 0)),
        out_shape=jax.ShapeDtypeStruct((_N, _D), jnp.float32),
    )(parts)


def _asm_body(x0_ref, x1_ref, x2_ref, p3_ref, o_ref):
    o_ref[:, 0, :] = x0_ref[...]
    o_ref[:, 1, :] = x1_ref[...]
    o_ref[:, 2, :] = x2_ref[...]
    o_ref[:, 3, :] = p3_ref[0] + p3_ref[1]


def _assemble(embs, parts3, base, rows):
    blk = 1000
    off = base // blk
    return pl.pallas_call(
        _asm_body,
        grid=(rows // blk,),
        in_specs=[pl.BlockSpec((blk, _D), lambda i, o=off: (i + o, 0))
                  for _ in range(_HOPS)] +
                 [pl.BlockSpec((_NC, blk, _D), lambda i, o=off: (0, i + o, 0))],
        out_specs=pl.BlockSpec((blk, _HOPS + 1, _D), lambda i: (i, 0, 0)),
        out_shape=jax.ShapeDtypeStruct((rows, _HOPS + 1, _D), jnp.float32),
    )(*embs, parts3)


def kernel(user_embed, item_embed, adj_indices, adj_values):
    x = jnp.concatenate([user_embed, item_embed], axis=0)
    pad = _EPAD - _E
    row = jnp.concatenate([adj_indices[0], jnp.zeros((pad,), jnp.int32)])
    col = jnp.concatenate([adj_indices[1], jnp.zeros((pad,), jnp.int32)])
    val = jnp.concatenate([adj_values, jnp.zeros((pad,), jnp.float32)])
    row = row.reshape(_NW, _NCH, _C)
    col = col.reshape(_NW, _NCH, _C)
    val = val.reshape(_NW, _NCH, _C)

    embs = [x]
    for _ in range(_HOPS - 1):
        parts = _hop(x, col, row, val)
        x = _combine(parts)
        embs.append(x)
    parts3 = _hop(x, col, row, val)
    return (_assemble(embs, parts3, 0, _N_USERS),
            _assemble(embs, parts3, _N_USERS, _N_ITEMS))
